# baseline (speedup 1.0000x reference)
import functools

import jax
import jax.numpy as jnp
from jax import lax
from jax.experimental import pallas as pl
from jax.experimental.pallas import tpu as pltpu
from jax.experimental.pallas import tpu_sc as plsc

_KC = 1024


def _vq_tc_kernel(x_ref, cb_ref, idx_ref, xt_s, x2_s, cmin_s):
    k = pl.program_id(0)
    B, D, T = x_ref.shape

    @pl.when(k == 0)
    def _():
        for b in range(B):
            xt_b = x_ref[b].T
            xt_s[pl.ds(b * T, T), :] = xt_b
            x2_s[pl.ds(b * T, T), :] = jnp.sum(xt_b ** 2, axis=-1,
                                               keepdims=True)

    cb = cb_ref[...]
    xt = xt_s[...]
    mm = jax.lax.dot_general(xt, cb, (((1,), (1,)), ((), ())),
                             preferred_element_type=jnp.float32)
    c2 = jnp.sum(cb ** 2, axis=-1)
    dist = x2_s[...] + c2[None, :] - 2.0 * mm
    lidx = jnp.argmin(dist, axis=1)
    lmin = jnp.min(dist, axis=1)
    gidx = k * _KC + lidx

    @pl.when(k == 0)
    def _():
        cmin_s[...] = lmin[None, :]
        idx_ref[...] = gidx[None, :]

    @pl.when(k > 0)
    def _():
        upd = lmin[None, :] < cmin_s[...]
        cmin_s[...] = jnp.where(upd, lmin[None, :], cmin_s[...])
        idx_ref[...] = jnp.where(upd, gidx[None, :], idx_ref[...])


def _make_sc_gather(n_rows, d, n_workers, nc):
    rows_per_w = n_rows // n_workers

    @functools.partial(
        pl.kernel,
        mesh=plsc.VectorSubcoreMesh(core_axis_name="c", subcore_axis_name="s"),
        out_type=jax.ShapeDtypeStruct((n_rows, d), jnp.float32),
        scratch_types=[
            pltpu.VMEM((rows_per_w,), jnp.int32),
            pltpu.VMEM((rows_per_w, d), jnp.float32),
            pltpu.SemaphoreType.DMA,
        ],
    )
    def sc_gather(table_hbm, idx_hbm, out_hbm, idx_v, rows_v, sem):
        wid = lax.axis_index("s") * nc + lax.axis_index("c")
        base = wid * rows_per_w
        pltpu.sync_copy(idx_hbm.at[pl.ds(base, rows_per_w)], idx_v)
        pltpu.async_copy(table_hbm.at[idx_v], rows_v, sem).wait()
        pltpu.sync_copy(rows_v, out_hbm.at[pl.ds(base, rows_per_w)])

    return sc_gather


def kernel(x, codebook):
    B, D, T = x.shape
    K = codebook.shape[0]
    BT = B * T
    idx2 = pl.pallas_call(
        _vq_tc_kernel,
        grid=(K // _KC,),
        in_specs=[pl.BlockSpec((B, D, T), lambda k: (0, 0, 0)),
                  pl.BlockSpec((_KC, D), lambda k: (k, 0))],
        out_specs=pl.BlockSpec((1, BT), lambda k: (0, 0)),
        out_shape=jax.ShapeDtypeStruct((1, BT), jnp.int32),
        scratch_shapes=[pltpu.VMEM((BT, D), jnp.float32),
                        pltpu.VMEM((BT, 1), jnp.float32),
                        pltpu.VMEM((1, BT), jnp.float32)],
    )(x, codebook)
    idx_flat = idx2.reshape(BT)
    info = plsc.get_sparse_core_info()
    nw = info.num_cores * info.num_subcores
    rows = _make_sc_gather(BT, D, nw, info.num_cores)(codebook, idx_flat)
    quantized = jnp.transpose(rows.reshape(B, T, D), (0, 2, 1))
    return quantized, idx_flat.reshape(B, T)


# speedup vs baseline: 1.2395x; 1.2395x over previous
import functools

import jax
import jax.numpy as jnp
from jax import lax
from jax.experimental import pallas as pl
from jax.experimental.pallas import tpu as pltpu
from jax.experimental.pallas import tpu_sc as plsc

_NC = 8


def _vq_tc_kernel(x_ref, cb_hbm, idx_ref, cb_v, dist_s, c2_s, sems):
    b = pl.program_id(0)
    K, D = cb_v.shape
    T = x_ref.shape[2]
    kc = K // _NC

    def _chunk_copy(c):
        return pltpu.make_async_copy(
            cb_hbm.at[pl.ds(c * kc, kc), :], cb_v.at[pl.ds(c * kc, kc), :],
            sems.at[c])

    @pl.when(b == 0)
    def _():
        for c in range(_NC):
            _chunk_copy(c).start()

    xt = x_ref[0].T
    x2 = jnp.sum(xt ** 2, axis=-1, keepdims=True)
    for c in range(_NC):
        @pl.when(b == 0)
        def _():
            _chunk_copy(c).wait()
        cbc = cb_v[pl.ds(c * kc, kc), :]

        @pl.when(b == 0)
        def _():
            c2_s[0, pl.ds(c * kc, kc)] = jnp.sum(cbc ** 2, axis=-1)
        mm = jax.lax.dot_general(xt, cbc, (((1,), (1,)), ((), ())),
                                 preferred_element_type=jnp.float32)
        c2 = c2_s[0, pl.ds(c * kc, kc)]
        dist_s[:, pl.ds(c * kc, kc)] = x2 + c2[None, :] - 2.0 * mm
    idx_ref[0, 0] = jnp.argmin(dist_s[...], axis=1)


def _make_sc_gather(n_rows, d, n_workers, nc):
    rows_per_w = n_rows // n_workers

    @functools.partial(
        pl.kernel,
        mesh=plsc.VectorSubcoreMesh(core_axis_name="c", subcore_axis_name="s"),
        out_type=jax.ShapeDtypeStruct((n_rows, d), jnp.float32),
        scratch_types=[
            pltpu.VMEM((rows_per_w,), jnp.int32),
            pltpu.VMEM((rows_per_w, d), jnp.float32),
            pltpu.SemaphoreType.DMA,
        ],
    )
    def sc_gather(table_hbm, idx_hbm, out_hbm, idx_v, rows_v, sem):
        wid = lax.axis_index("s") * nc + lax.axis_index("c")
        base = wid * rows_per_w
        pltpu.sync_copy(idx_hbm.at[pl.ds(base, rows_per_w)], idx_v)
        pltpu.async_copy(table_hbm.at[idx_v], rows_v, sem).wait()
        pltpu.sync_copy(rows_v, out_hbm.at[pl.ds(base, rows_per_w)])

    return sc_gather


def kernel(x, codebook):
    B, D, T = x.shape
    K = codebook.shape[0]
    idx3 = pl.pallas_call(
        _vq_tc_kernel,
        grid=(B,),
        in_specs=[pl.BlockSpec((1, D, T), lambda b: (b, 0, 0)),
                  pl.BlockSpec(memory_space=pl.ANY)],
        out_specs=pl.BlockSpec((1, 1, T), lambda b: (b, 0, 0)),
        out_shape=jax.ShapeDtypeStruct((B, 1, T), jnp.int32),
        scratch_shapes=[pltpu.VMEM((K, D), jnp.float32),
                        pltpu.VMEM((T, K), jnp.float32),
                        pltpu.VMEM((1, K), jnp.float32),
                        pltpu.SemaphoreType.DMA((_NC,))],
    )(x, codebook)
    idx_flat = idx3.reshape(B * T)
    info = plsc.get_sparse_core_info()
    nw = info.num_cores * info.num_subcores
    rows = _make_sc_gather(B * T, D, nw, info.num_cores)(codebook, idx_flat)
    quantized = jnp.transpose(rows.reshape(B, T, D), (0, 2, 1))
    return quantized, idx_flat.reshape(B, T)


# speedup vs baseline: 1.5275x; 1.2323x over previous
import functools

import jax
import jax.numpy as jnp
from jax import lax
from jax.experimental import pallas as pl
from jax.experimental.pallas import tpu as pltpu
from jax.experimental.pallas import tpu_sc as plsc

_NC = 8


def _vq_tc_kernel(x_ref, cb_hbm, idx_ref, cb_v, sems):
    b = pl.program_id(0)
    K, D = cb_v.shape
    kc = K // _NC

    def _chunk_copy(c):
        return pltpu.make_async_copy(
            cb_hbm.at[pl.ds(c * kc, kc), :], cb_v.at[pl.ds(c * kc, kc), :],
            sems.at[c])

    @pl.when(b == 0)
    def _():
        for c in range(_NC):
            _chunk_copy(c).start()
        for c in range(_NC):
            _chunk_copy(c).wait()

    xt = x_ref[0].T
    cb = cb_v[...]
    mm = jax.lax.dot_general(xt, cb, (((1,), (1,)), ((), ())),
                             preferred_element_type=jnp.float32)
    x2 = jnp.sum(xt ** 2, axis=-1, keepdims=True)
    c2 = jnp.sum(cb ** 2, axis=-1)
    dist = x2 + c2[None, :] - 2.0 * mm
    idx_ref[0, 0] = jnp.argmin(dist, axis=1)


def _make_sc_gather(n_rows, d, n_workers, nc):
    rows_per_w = n_rows // n_workers

    @functools.partial(
        pl.kernel,
        mesh=plsc.VectorSubcoreMesh(core_axis_name="c", subcore_axis_name="s"),
        out_type=jax.ShapeDtypeStruct((n_rows, d), jnp.float32),
        scratch_types=[
            pltpu.VMEM((rows_per_w,), jnp.int32),
            pltpu.VMEM((rows_per_w, d), jnp.float32),
            pltpu.SemaphoreType.DMA,
        ],
    )
    def sc_gather(table_hbm, idx_hbm, out_hbm, idx_v, rows_v, sem):
        wid = lax.axis_index("s") * nc + lax.axis_index("c")
        base = wid * rows_per_w
        pltpu.sync_copy(idx_hbm.at[pl.ds(base, rows_per_w)], idx_v)
        pltpu.async_copy(table_hbm.at[idx_v], rows_v, sem).wait()
        pltpu.sync_copy(rows_v, out_hbm.at[pl.ds(base, rows_per_w)])

    return sc_gather


def kernel(x, codebook):
    B, D, T = x.shape
    K = codebook.shape[0]
    idx3 = pl.pallas_call(
        _vq_tc_kernel,
        grid=(B,),
        in_specs=[pl.BlockSpec((1, D, T), lambda b: (b, 0, 0)),
                  pl.BlockSpec(memory_space=pl.ANY)],
        out_specs=pl.BlockSpec((1, 1, T), lambda b: (b, 0, 0)),
        out_shape=jax.ShapeDtypeStruct((B, 1, T), jnp.int32),
        scratch_shapes=[pltpu.VMEM((K, D), jnp.float32),
                        pltpu.SemaphoreType.DMA((_NC,))],
    )(x, codebook)
    idx_flat = idx3.reshape(B * T)
    info = plsc.get_sparse_core_info()
    nw = info.num_cores * info.num_subcores
    rows = _make_sc_gather(B * T, D, nw, info.num_cores)(codebook, idx_flat)
    quantized = jnp.transpose(rows.reshape(B, T, D), (0, 2, 1))
    return quantized, idx_flat.reshape(B, T)
